# baseline (speedup 1.0000x reference)
import jax
import jax.numpy as jnp
from jax import lax
from jax.experimental import pallas as pl
from jax.experimental.pallas import tpu as pltpu
from jax.experimental.pallas import tpu_sc as plsc

VOCAB = 1000000
LSEQ = 200
D = 64
BATCH = 4096

NC = 2
NS = 16
NW = NC * NS

WTILES = BATCH // 128
NBUF = 4


def _sc_body(tok_hbm, xi_hbm, pos_hbm, out_hbm,
             xall, g0, g1, g2, g3, o0, o1, o2, o3, pos_v,
             gsem0, gsem1, gsem2, gsem3, ssem0, ssem1, ssem2, ssem3):
    gbuf = (g0, g1, g2, g3)
    obuf = (o0, o1, o2, o3)
    gsem = (gsem0, gsem1, gsem2, gsem3)
    ssem = (ssem0, ssem1, ssem2, ssem3)

    w = lax.axis_index("s") * NC + lax.axis_index("c")

    pltpu.sync_copy(pos_hbm, pos_v)
    pltpu.sync_copy(xi_hbm.at[:, w], xall)

    def fetch(b, l):
        pltpu.async_copy(tok_hbm.at[xall.at[l]], gbuf[b], gsem[b])

    for b in range(NBUF):
        fetch(b, b)

    iota = lax.iota(jnp.int32, 16)

    @pl.loop(0, LSEQ // NBUF)
    def _grp(t):
        for b in range(NBUF):
            l = t * NBUF + b
            pltpu.make_async_copy(tok_hbm.at[pl.ds(0, 128)], gbuf[b],
                                  gsem[b]).wait()

            @pl.when(t > 0)
            def _():
                pltpu.make_async_copy(
                    obuf[b], out_hbm.at[0, :, 0], ssem[b]).wait()

            pr = l // 2
            pc = (l % 2) * 64

            @pl.loop(0, 8)
            def _g(g):
                for s in range(8):
                    f = g * 8 + s
                    fv = jnp.full((16,), f, jnp.int32)
                    ps = plsc.load_gather(
                        pos_v, [jnp.full((16,), pr, jnp.int32),
                                jnp.full((16,), pc, jnp.int32) + fv])
                    vals = [plsc.load_gather(gbuf[b], [iota + (jg * 16), fv])
                            for jg in range(8)]
                    for jg in range(8):
                        obuf[b][g, s, pl.ds(jg * 16, 16)] = vals[jg] + ps

            pltpu.async_copy(obuf[b], out_hbm.at[l, :, w], ssem[b])

            @pl.when(l + NBUF < LSEQ)
            def _():
                fetch(b, l + NBUF)

    for b in range(NBUF):
        pltpu.make_async_copy(obuf[b], out_hbm.at[0, :, 0], ssem[b]).wait()


@jax.jit
def _sc_embed(tok, xi3, pos2):
    mesh = plsc.VectorSubcoreMesh(core_axis_name="c", subcore_axis_name="s")
    fn = pl.kernel(
        _sc_body,
        out_type=jax.ShapeDtypeStruct((LSEQ, 8, WTILES, 8, 128), jnp.float32),
        mesh=mesh,
        scratch_types=[
            pltpu.VMEM((LSEQ, 128), jnp.int32),
            pltpu.VMEM((128, D), jnp.float32),
            pltpu.VMEM((128, D), jnp.float32),
            pltpu.VMEM((128, D), jnp.float32),
            pltpu.VMEM((128, D), jnp.float32),
            pltpu.VMEM((8, 8, 128), jnp.float32),
            pltpu.VMEM((8, 8, 128), jnp.float32),
            pltpu.VMEM((8, 8, 128), jnp.float32),
            pltpu.VMEM((8, 8, 128), jnp.float32),
            pltpu.VMEM((100, 128), jnp.float32),
            pltpu.SemaphoreType.DMA,
            pltpu.SemaphoreType.DMA,
            pltpu.SemaphoreType.DMA,
            pltpu.SemaphoreType.DMA,
            pltpu.SemaphoreType.DMA,
            pltpu.SemaphoreType.DMA,
            pltpu.SemaphoreType.DMA,
            pltpu.SemaphoreType.DMA,
        ],
        compiler_params=pltpu.CompilerParams(use_tc_tiling_on_sc=False,
                                             needs_layout_passes=False),
    )
    return fn(tok, xi3, pos2)


def kernel(x, token_table, pos_table):
    xi3 = x.astype(jnp.int32).T.reshape(LSEQ, WTILES, 128)
    pos2 = pos_table.reshape(100, 128)
    out5 = _sc_embed(token_table, xi3, pos2)
    return out5.transpose(2, 4, 0, 1, 3).reshape(BATCH, LSEQ, D)


# speedup vs baseline: 1.4485x; 1.4485x over previous
import jax
import jax.numpy as jnp
from jax import lax
from jax.experimental import pallas as pl
from jax.experimental.pallas import tpu as pltpu
from jax.experimental.pallas import tpu_sc as plsc

VOCAB = 1000000
LSEQ = 200
D = 64
BATCH = 4096

NC = 2
NS = 16
NW = NC * NS

WTILES = BATCH // 128
NBUF = 4


def _sc_body(tok_hbm, xi_hbm, pos_hbm, out_hbm,
             xall, g0, g1, g2, g3, o0, o1, o2, o3, pos_v,
             gsem0, gsem1, gsem2, gsem3, ssem0, ssem1, ssem2, ssem3):
    gbuf = (g0, g1, g2, g3)
    obuf = (o0, o1, o2, o3)
    gsem = (gsem0, gsem1, gsem2, gsem3)
    ssem = (ssem0, ssem1, ssem2, ssem3)

    w = lax.axis_index("s") * NC + lax.axis_index("c")

    pltpu.sync_copy(pos_hbm, pos_v)
    pltpu.sync_copy(xi_hbm.at[:, w], xall)

    def fetch(b, l):
        pltpu.async_copy(tok_hbm.at[xall.at[l]], gbuf[b], gsem[b])

    for b in range(NBUF):
        fetch(b, b)

    iota = lax.iota(jnp.int32, 16)

    @pl.loop(0, LSEQ // NBUF)
    def _grp(t):
        for b in range(NBUF):
            l = t * NBUF + b
            pltpu.make_async_copy(tok_hbm.at[pl.ds(0, 128)], gbuf[b],
                                  gsem[b]).wait()

            @pl.when(t > 0)
            def _():
                pltpu.make_async_copy(
                    obuf[b].at[:, :, pl.ds(0, 128)],
                    out_hbm.at[0, :, 0], ssem[b]).wait()

            pr = l // 2
            pc = (l % 2) * 64

            posv = [pos_v[pr, pl.ds(pc + e * 16, 16)] for e in range(4)]
            gv = iota >> 3
            sv = iota & 7

            for e in range(4):
                gvec = gv + (e * 2)
                pse = posv[e]

                @pl.loop(0, 128, unroll=4)
                def _j(j):
                    val = gbuf[b][j, pl.ds(e * 16, 16)] + pse
                    plsc.store_scatter(
                        obuf[b], [gvec, sv, jnp.full((16,), j, jnp.int32)],
                        val)

            pltpu.async_copy(obuf[b].at[:, :, pl.ds(0, 128)],
                             out_hbm.at[l, :, w], ssem[b])

            @pl.when(l + NBUF < LSEQ)
            def _():
                fetch(b, l + NBUF)

    for b in range(NBUF):
        pltpu.make_async_copy(obuf[b].at[:, :, pl.ds(0, 128)],
                              out_hbm.at[0, :, 0], ssem[b]).wait()


@jax.jit
def _sc_embed(tok, xi3, pos2):
    mesh = plsc.VectorSubcoreMesh(core_axis_name="c", subcore_axis_name="s")
    fn = pl.kernel(
        _sc_body,
        out_type=jax.ShapeDtypeStruct((LSEQ, 8, WTILES, 8, 128), jnp.float32),
        mesh=mesh,
        scratch_types=[
            pltpu.VMEM((LSEQ, 128), jnp.int32),
            pltpu.VMEM((128, D), jnp.float32),
            pltpu.VMEM((128, D), jnp.float32),
            pltpu.VMEM((128, D), jnp.float32),
            pltpu.VMEM((128, D), jnp.float32),
            pltpu.VMEM((8, 8, 129), jnp.float32),
            pltpu.VMEM((8, 8, 129), jnp.float32),
            pltpu.VMEM((8, 8, 129), jnp.float32),
            pltpu.VMEM((8, 8, 129), jnp.float32),
            pltpu.VMEM((100, 128), jnp.float32),
            pltpu.SemaphoreType.DMA,
            pltpu.SemaphoreType.DMA,
            pltpu.SemaphoreType.DMA,
            pltpu.SemaphoreType.DMA,
            pltpu.SemaphoreType.DMA,
            pltpu.SemaphoreType.DMA,
            pltpu.SemaphoreType.DMA,
            pltpu.SemaphoreType.DMA,
        ],
        compiler_params=pltpu.CompilerParams(use_tc_tiling_on_sc=False,
                                             needs_layout_passes=False),
    )
    return fn(tok, xi3, pos2)


def kernel(x, token_table, pos_table):
    xi3 = x.astype(jnp.int32).T.reshape(LSEQ, WTILES, 128)
    pos2 = pos_table.reshape(100, 128)
    out5 = _sc_embed(token_table, xi3, pos2)
    return out5.transpose(2, 4, 0, 1, 3).reshape(BATCH, LSEQ, D)
